# Initial kernel scaffold; baseline (speedup 1.0000x reference)
import jax
import jax.numpy as jnp
from jax.experimental import pallas as pl


def kernel(X, W):
    raise NotImplementedError("write your pallas kernel here")


# speedup vs baseline: 1.5128x; 1.5128x over previous
import functools

import jax
import jax.numpy as jnp
from jax import lax
from jax.experimental import pallas as pl
from jax.experimental.pallas import tpu as pltpu
from jax.experimental.pallas import tpu_sc as plsc

NC = 2
NS = 16
NW = NC * NS

B_TOTAL = 1024 * 26 * 20
D = 32
B_PER_W = B_TOTAL // NW
CHUNK = 1664
NCHUNK = B_PER_W // CHUNK


@functools.partial(
    pl.kernel,
    out_type=jax.ShapeDtypeStruct((B_TOTAL, D), jnp.float32),
    mesh=plsc.VectorSubcoreMesh(core_axis_name="c", subcore_axis_name="s"),
    scratch_types=[
        pltpu.VMEM((CHUNK,), jnp.int32),
        pltpu.VMEM((CHUNK, D), jnp.float32),
        pltpu.SemaphoreType.DMA,
    ],
    compiler_params=pltpu.CompilerParams(use_tc_tiling_on_sc=False),
)
def _emb_gather(idx_hbm, table_hbm, out_hbm, idx_v, rows_v, sem):
    wid = lax.axis_index("s") * NC + lax.axis_index("c")
    base = wid * B_PER_W
    for c in range(NCHUNK):
        off = base + c * CHUNK
        pltpu.sync_copy(idx_hbm.at[pl.ds(off, CHUNK)], idx_v)
        pltpu.async_copy(table_hbm.at[idx_v], rows_v, sem).wait()
        pltpu.sync_copy(rows_v, out_hbm.at[pl.ds(off, CHUNK)])


def kernel(X, W):
    idx = X.reshape(-1).astype(jnp.int32)
    out = _emb_gather(idx, W)
    return out.reshape(X.shape + (W.shape[1],))


# speedup vs baseline: 1.5310x; 1.0120x over previous
import functools

import jax
import jax.numpy as jnp
from jax import lax
from jax.experimental import pallas as pl
from jax.experimental.pallas import tpu as pltpu
from jax.experimental.pallas import tpu_sc as plsc

NC = 2
NS = 16
NW = NC * NS

B_TOTAL = 1024 * 26 * 20
D = 32
B_PER_W = B_TOTAL // NW
CHUNK = 1664
NCHUNK = B_PER_W // CHUNK


@functools.partial(
    pl.kernel,
    out_type=jax.ShapeDtypeStruct((B_TOTAL, D), jnp.float32),
    mesh=plsc.VectorSubcoreMesh(core_axis_name="c", subcore_axis_name="s"),
    scratch_types=[
        pltpu.VMEM((B_PER_W,), jnp.int32),
        pltpu.VMEM((CHUNK, D), jnp.float32),
        pltpu.VMEM((CHUNK, D), jnp.float32),
        pltpu.SemaphoreType.DMA,
        pltpu.SemaphoreType.DMA,
        pltpu.SemaphoreType.DMA,
        pltpu.SemaphoreType.DMA,
    ],
    compiler_params=pltpu.CompilerParams(use_tc_tiling_on_sc=False),
)
def _emb_gather(idx_hbm, table_hbm, out_hbm, idx_v, rows0, rows1,
                sg0, sg1, so0, so1):
    wid = lax.axis_index("s") * NC + lax.axis_index("c")
    base = wid * B_PER_W
    pltpu.sync_copy(idx_hbm.at[pl.ds(base, B_PER_W)], idx_v)
    rows = (rows0, rows1)
    sg = (sg0, sg1)
    so = (so0, so1)
    gathers = [None] * NCHUNK
    outs = [None] * NCHUNK
    for c in range(NCHUNK):
        b = c % 2
        if c >= 2:
            outs[c - 2].wait()
        gathers[c] = pltpu.async_copy(
            table_hbm.at[idx_v.at[pl.ds(c * CHUNK, CHUNK)]], rows[b], sg[b])
        if c >= 1:
            gathers[c - 1].wait()
            outs[c - 1] = pltpu.async_copy(
                rows[1 - b], out_hbm.at[pl.ds(base + (c - 1) * CHUNK, CHUNK)],
                so[1 - b])
    last = NCHUNK - 1
    gathers[last].wait()
    outs[last] = pltpu.async_copy(
        rows[last % 2], out_hbm.at[pl.ds(base + last * CHUNK, CHUNK)],
        so[last % 2])
    outs[last - 1].wait()
    outs[last].wait()


def kernel(X, W):
    idx = X.reshape(-1).astype(jnp.int32)
    out = _emb_gather(idx, W)
    return out.reshape(X.shape + (W.shape[1],))
